# Initial kernel scaffold; baseline (speedup 1.0000x reference)
import jax
import jax.numpy as jnp
from jax.experimental import pallas as pl


def kernel(index, index_len, x, x_teature, gamma, Wq, bq, Wk, bk, Wv, bv):
    raise NotImplementedError("write your pallas kernel here")


# speedup vs baseline: 9.9104x; 9.9104x over previous
import functools

import jax
import jax.numpy as jnp
from jax import lax
from jax.experimental import pallas as pl
from jax.experimental.pallas import tpu as pltpu
from jax.experimental.pallas import tpu_sc as plsc
from jax._src.pallas import mpmd as _plmpmd

_NC = 2
_NS = 16
_NW = _NC * _NS
_ROWS = 13
_GW = _NW * _ROWS * 128

_NT = 256
_HWT = 3584
_QW = 24


def _sc_mesh():
  return plsc.VectorSubcoreMesh(
      core_axis_name="c", subcore_axis_name="s",
      num_cores=_NC, num_subcores=_NS)


def _sc_gather(table, gidx3):
  @functools.partial(
      pl.kernel,
      out_type=jax.ShapeDtypeStruct((_NW, _ROWS, 128), jnp.float32),
      mesh=_sc_mesh(),
      scratch_types=[
          pltpu.VMEM((_ROWS, 128), jnp.int32),
          pltpu.VMEM((_ROWS, 128), jnp.float32),
          pltpu.SemaphoreType.DMA,
      ],
  )
  def gk(table_hbm, idx_hbm, out_hbm, idx_v, rows_v, sem):
    wid = lax.axis_index("s") * _NC + lax.axis_index("c")
    pltpu.sync_copy(idx_hbm.at[wid], idx_v)
    copies = [
        pltpu.async_copy(table_hbm.at[idx_v.at[j]], rows_v.at[j], sem)
        for j in range(_ROWS)
    ]
    for c in copies:
      c.wait()
    pltpu.sync_copy(rows_v, out_hbm.at[wid])

  return gk(table, gidx3)


def _sc_scatter(y0, sidx3, supd3):
  def sk(y_in_hbm, idx_hbm, val_hbm, out_hbm, idx_v, val_v, sem):
    del y_in_hbm
    wid = lax.axis_index("s") * _NC + lax.axis_index("c")
    pltpu.sync_copy(idx_hbm.at[wid], idx_v)
    pltpu.sync_copy(val_hbm.at[wid], val_v)
    copies = [
        pltpu.async_copy(val_v.at[j], out_hbm.at[idx_v.at[j]], sem)
        for j in range(_ROWS)
    ]
    for c in copies:
      c.wait()

  fn = _plmpmd._mpmd_map(
      [(_sc_mesh(), sk)],
      jax.ShapeDtypeStruct(y0.shape, jnp.float32),
      input_output_aliases={0: 0},
      scratch_types=[
          pltpu.VMEM((_ROWS, 128), jnp.int32),
          pltpu.VMEM((_ROWS, 128), jnp.float32),
          pltpu.SemaphoreType.DMA,
      ],
  )
  return fn(y0, sidx3, supd3)


def _flash_body(xg_ref, b_ref, gv_ref, xf_ref, xt_ref, blk_ref, wqt_ref,
                wk_ref, sexp_ref, ssel_ref, wvt_ref, bq_ref, bv_ref,
                out_ref, qe_ref, m_ref, l_ref, acc_ref, *, nh):
  j = pl.program_id(1)

  @pl.when(j == 0)
  def _init():
    q = jnp.dot(xg_ref[...], wqt_ref[...],
                preferred_element_type=jnp.float32) + bq_ref[0:1, :]
    qk = jnp.dot(q, wk_ref[...], preferred_element_type=jnp.float32)
    mask = (blk_ref[0:1, :] == b_ref[...]).astype(jnp.float32)
    qe_ref[...] = jnp.dot(qk, sexp_ref[...],
                          preferred_element_type=jnp.float32) * mask
    m_ref[...] = jnp.full_like(m_ref[...], -jnp.inf)
    l_ref[...] = jnp.zeros_like(l_ref[...])
    acc_ref[...] = jnp.zeros_like(acc_ref[...])

  e = jnp.dot(qe_ref[...], xf_ref[...], preferred_element_type=jnp.float32)
  cm = jnp.max(e, axis=1, keepdims=True)
  m_new = jnp.maximum(m_ref[...], cm)
  alpha = jnp.exp(m_ref[...] - m_new)
  p = jnp.exp(e - m_new)
  l_ref[...] = l_ref[...] * alpha + jnp.sum(p, axis=1, keepdims=True)
  acc_ref[...] = acc_ref[...] * alpha + jnp.dot(
      p, xt_ref[...], preferred_element_type=jnp.float32)
  m_ref[...] = m_new

  @pl.when(j == nh - 1)
  def _fin():
    mask = (blk_ref[0:1, :] == b_ref[...]).astype(jnp.float32)
    aexp = acc_ref[...] / l_ref[...]
    asel = jnp.dot(aexp * mask, ssel_ref[...],
                   preferred_element_type=jnp.float32)
    outv = jnp.dot(asel, wvt_ref[...],
                   preferred_element_type=jnp.float32) + bv_ref[0:1, :]
    out_ref[...] = xg_ref[...] + gv_ref[...] * outv


def _tc_flash(xg8, bvec, gv, xf24, xt24, blk8, wqt8, wk8, sexp, ssel, wvt8,
              bq8, bv8, np_, hw):
  nn = np_ // _NT
  nh = hw // _HWT
  grid = (nn, nh)
  return pl.pallas_call(
      functools.partial(_flash_body, nh=nh),
      grid=grid,
      in_specs=[
          pl.BlockSpec((_NT, 8), lambda i, j: (i, 0)),
          pl.BlockSpec((_NT, 1), lambda i, j: (i, 0)),
          pl.BlockSpec((_NT, 1), lambda i, j: (i, 0)),
          pl.BlockSpec((_QW, _HWT), lambda i, j: (0, j)),
          pl.BlockSpec((_HWT, _QW), lambda i, j: (j, 0)),
          pl.BlockSpec((8, _QW), lambda i, j: (0, 0)),
          pl.BlockSpec((8, 8), lambda i, j: (0, 0)),
          pl.BlockSpec((8, 8), lambda i, j: (0, 0)),
          pl.BlockSpec((8, _QW), lambda i, j: (0, 0)),
          pl.BlockSpec((_QW, 8), lambda i, j: (0, 0)),
          pl.BlockSpec((8, 8), lambda i, j: (0, 0)),
          pl.BlockSpec((8, 8), lambda i, j: (0, 0)),
          pl.BlockSpec((8, 8), lambda i, j: (0, 0)),
      ],
      out_specs=pl.BlockSpec((_NT, 8), lambda i, j: (i, 0)),
      out_shape=jax.ShapeDtypeStruct((np_, 8), jnp.float32),
      scratch_shapes=[
          pltpu.VMEM((_NT, _QW), jnp.float32),
          pltpu.VMEM((_NT, 1), jnp.float32),
          pltpu.VMEM((_NT, 1), jnp.float32),
          pltpu.VMEM((_NT, _QW), jnp.float32),
      ],
  )(xg8, bvec, gv, xf24, xt24, blk8, wqt8, wk8, sexp, ssel, wvt8, bq8, bv8)


def _pad8(w):
  return jnp.pad(w, ((0, 3), (0, 3)))


def kernel(index, index_len, x, x_teature, gamma, Wq, bq, Wk, bk, Wv, bv):
  del x_teature, bk
  B, C, H, W = x.shape
  HW = H * W
  BCHW = B * C * HW
  N = index.shape[0]
  NP = ((N + _NT - 1) // _NT) * _NT

  b_i = index[:, 0].astype(jnp.int32)
  p_i = (index[:, 1] * W + index[:, 2]).astype(jnp.int32)
  word = (b_i * C)[:, None] * HW + jnp.arange(C, dtype=jnp.int32)[None, :] * HW \
      + p_i[:, None]
  wflat = word.reshape(N * C)

  gidx = jnp.concatenate(
      [wflat, jnp.zeros((_GW - N * C,), jnp.int32)]).reshape(_NW, _ROWS, 128)
  xg_words = _sc_gather(x.reshape(BCHW), gidx).reshape(_GW)
  xg = xg_words[:N * C].reshape(N, C)
  xg8 = jnp.pad(xg, ((0, NP - N), (0, 8 - C)))

  nvalid = jnp.minimum(index_len, N)
  gv = jnp.where(jnp.arange(NP) < nvalid, gamma, 0.0).astype(
      jnp.float32)[:, None]
  bvec = jnp.pad(b_i, (0, NP - N))[:, None]
  xf24 = jnp.pad(x.reshape(B * C, HW), ((0, _QW - B * C), (0, 0)))
  xt24 = xf24.T
  blk8 = jnp.tile(
      jnp.concatenate([jnp.repeat(jnp.arange(B, dtype=jnp.int32), C),
                       jnp.full((_QW - B * C,), 99, jnp.int32)])[None, :],
      (8, 1))
  sexp = jnp.concatenate(
      [jnp.pad(jnp.eye(C, dtype=jnp.float32), ((0, 8 - C), (0, 0)))] * B,
      axis=1)
  sexp = jnp.pad(sexp, ((0, 0), (0, _QW - B * C)))
  ssel = sexp.T
  bq8 = jnp.tile(jnp.pad(bq, (0, 8 - C))[None, :], (8, 1))
  bv8 = jnp.tile(jnp.pad(bv, (0, 8 - C))[None, :], (8, 1))
  upd = _tc_flash(xg8, bvec, gv, xf24, xt24, blk8, _pad8(Wq.T), _pad8(Wk),
                  sexp, ssel, _pad8(Wv.T), bq8, bv8, NP, HW)

  supd = jnp.concatenate(
      [upd[:, :C].reshape(NP * C),
       jnp.zeros((_GW - NP * C,), jnp.float32)]).reshape(_NW, _ROWS, 128)
  sidx = jnp.concatenate(
      [wflat, jnp.full((_GW - N * C,), BCHW, jnp.int32)]).reshape(
          _NW, _ROWS, 128)
  y0 = jnp.concatenate([x.reshape(BCHW), jnp.zeros((128,), jnp.float32)])
  yext = _sc_scatter(y0, sidx, supd)
  y = yext[:BCHW].reshape(B, C, H, W)
  return (y, y)


# speedup vs baseline: 10.4460x; 1.0540x over previous
import functools

import jax
import jax.numpy as jnp
from jax import lax
from jax.experimental import pallas as pl
from jax.experimental.pallas import tpu as pltpu
from jax.experimental.pallas import tpu_sc as plsc
from jax._src.pallas import mpmd as _plmpmd

_NC = 2
_NS = 16
_NW = _NC * _NS
_ROWS = 3
_NG = _NW * _ROWS * 128
_LN = 16
_SPARE = 2304

_NT = 256
_HWT = 3584
_QW = 24


def _sc_mesh():
  return plsc.VectorSubcoreMesh(
      core_axis_name="c", subcore_axis_name="s",
      num_cores=_NC, num_subcores=_NS)


def _sc_gather(table, gidx3):
  @functools.partial(
      pl.kernel,
      out_type=jax.ShapeDtypeStruct((_NW, _ROWS, 128, _LN), jnp.float32),
      mesh=_sc_mesh(),
      compiler_params=pltpu.CompilerParams(use_tc_tiling_on_sc=False),
      scratch_types=[
          pltpu.VMEM((_ROWS, 128), jnp.int32),
          pltpu.VMEM((_ROWS, 128, _LN), jnp.float32),
          pltpu.SemaphoreType.DMA,
      ],
  )
  def gk(table_hbm, idx_hbm, out_hbm, idx_v, rows_v, sem):
    wid = lax.axis_index("s") * _NC + lax.axis_index("c")
    pltpu.sync_copy(idx_hbm.at[wid], idx_v)
    copies = [
        pltpu.async_copy(table_hbm.at[idx_v.at[j]], rows_v.at[j], sem)
        for j in range(_ROWS)
    ]
    for c in copies:
      c.wait()
    pltpu.sync_copy(rows_v, out_hbm.at[wid])

  return gk(table, gidx3)


def _sc_scatter(y0, sidx3, supd3):
  def sk(y_in_hbm, idx_hbm, val_hbm, out_hbm, idx_v, val_v, sem):
    del y_in_hbm
    wid = lax.axis_index("s") * _NC + lax.axis_index("c")
    pltpu.sync_copy(idx_hbm.at[wid], idx_v)
    pltpu.sync_copy(val_hbm.at[wid], val_v)
    copies = [
        pltpu.async_copy(val_v.at[j], out_hbm.at[idx_v.at[j]], sem)
        for j in range(_ROWS)
    ]
    for c in copies:
      c.wait()

  fn = _plmpmd._mpmd_map(
      [(_sc_mesh(), sk)],
      jax.ShapeDtypeStruct(y0.shape, jnp.float32),
      input_output_aliases={0: 0},
      compiler_params=pltpu.CompilerParams(use_tc_tiling_on_sc=False),
      scratch_types=[
          pltpu.VMEM((_ROWS, 128), jnp.int32),
          pltpu.VMEM((_ROWS, 128, _LN), jnp.float32),
          pltpu.SemaphoreType.DMA,
      ],
  )
  return fn(y0, sidx3, supd3)


def _flash_body(xg_ref, b_ref, gv_ref, xf_ref, xt_ref, blk_ref, wqt_ref,
                wk_ref, sexp_ref, ssel_ref, wvt_ref, bq_ref, bv_ref,
                out_ref, qe_ref, m_ref, l_ref, acc_ref, *, nh):
  j = pl.program_id(1)

  @pl.when(j == 0)
  def _init():
    q = jnp.dot(xg_ref[...], wqt_ref[...],
                preferred_element_type=jnp.float32) + bq_ref[0:1, :]
    qk = jnp.dot(q, wk_ref[...], preferred_element_type=jnp.float32)
    mask = (blk_ref[0:1, :] == b_ref[...]).astype(jnp.float32)
    qe_ref[...] = jnp.dot(qk, sexp_ref[...],
                          preferred_element_type=jnp.float32) * mask
    m_ref[...] = jnp.full_like(m_ref[...], -jnp.inf)
    l_ref[...] = jnp.zeros_like(l_ref[...])
    acc_ref[...] = jnp.zeros_like(acc_ref[...])

  e = jnp.dot(qe_ref[...].astype(jnp.bfloat16), xf_ref[...],
              preferred_element_type=jnp.float32)
  cm = jnp.max(e, axis=1, keepdims=True)
  m_new = jnp.maximum(m_ref[...], cm)
  alpha = jnp.exp(m_ref[...] - m_new)
  p = jnp.exp(e - m_new)
  l_ref[...] = l_ref[...] * alpha + jnp.sum(p, axis=1, keepdims=True)
  acc_ref[...] = acc_ref[...] * alpha + jnp.dot(
      p.astype(jnp.bfloat16), xt_ref[...], preferred_element_type=jnp.float32)
  m_ref[...] = m_new

  @pl.when(j == nh - 1)
  def _fin():
    mask = (blk_ref[0:1, :] == b_ref[...]).astype(jnp.float32)
    aexp = acc_ref[...] / l_ref[...]
    asel = jnp.dot(aexp * mask, ssel_ref[...],
                   preferred_element_type=jnp.float32)
    outv = jnp.dot(asel, wvt_ref[...],
                   preferred_element_type=jnp.float32) + bv_ref[0:1, :]
    out_ref[...] = xg_ref[...] + gv_ref[...] * outv


def _tc_flash(xg16, bvec, gv, xf24, xt24, blk8, wqt, wk, sexp, ssel, wvt,
              bq16, bv16, np_, hw):
  nn = np_ // _NT
  nh = hw // _HWT
  grid = (nn, nh)
  return pl.pallas_call(
      functools.partial(_flash_body, nh=nh),
      grid=grid,
      in_specs=[
          pl.BlockSpec((_NT, _LN), lambda i, j: (i, 0)),
          pl.BlockSpec((_NT, 1), lambda i, j: (i, 0)),
          pl.BlockSpec((_NT, 1), lambda i, j: (i, 0)),
          pl.BlockSpec((_QW, _HWT), lambda i, j: (0, j)),
          pl.BlockSpec((_HWT, _QW), lambda i, j: (j, 0)),
          pl.BlockSpec((8, _QW), lambda i, j: (0, 0)),
          pl.BlockSpec((_LN, _LN), lambda i, j: (0, 0)),
          pl.BlockSpec((_LN, _LN), lambda i, j: (0, 0)),
          pl.BlockSpec((_LN, _QW), lambda i, j: (0, 0)),
          pl.BlockSpec((_QW, _LN), lambda i, j: (0, 0)),
          pl.BlockSpec((_LN, _LN), lambda i, j: (0, 0)),
          pl.BlockSpec((8, _LN), lambda i, j: (0, 0)),
          pl.BlockSpec((8, _LN), lambda i, j: (0, 0)),
      ],
      out_specs=pl.BlockSpec((_NT, _LN), lambda i, j: (i, 0)),
      out_shape=jax.ShapeDtypeStruct((np_, _LN), jnp.float32),
      scratch_shapes=[
          pltpu.VMEM((_NT, _QW), jnp.float32),
          pltpu.VMEM((_NT, 1), jnp.float32),
          pltpu.VMEM((_NT, 1), jnp.float32),
          pltpu.VMEM((_NT, _QW), jnp.float32),
      ],
  )(xg16, bvec, gv, xf24, xt24, blk8, wqt, wk, sexp, ssel, wvt, bq16, bv16)


def _padw(w):
  return jnp.pad(w, ((0, _LN - w.shape[0]), (0, _LN - w.shape[1])))


def kernel(index, index_len, x, x_teature, gamma, Wq, bq, Wk, bk, Wv, bv):
  del x_teature, bk
  B, C, H, W = x.shape
  HW = H * W
  BHW = B * HW
  N = index.shape[0]
  NP = ((N + _NT - 1) // _NT) * _NT

  b_i = index[:, 0].astype(jnp.int32)
  p_i = (index[:, 1] * W + index[:, 2]).astype(jnp.int32)
  rowidx = b_i * HW + p_i

  xperm = jnp.pad(
      x.reshape(B, C, HW).transpose(0, 2, 1).reshape(BHW, C),
      ((0, 0), (0, _LN - C)))

  gidx = jnp.concatenate(
      [rowidx, jnp.arange(_NG - N, dtype=jnp.int32) % BHW]).reshape(
          _NW, _ROWS, 128)
  xg16 = _sc_gather(xperm, gidx).reshape(_NG, _LN)[:NP]

  nvalid = jnp.minimum(index_len, N)
  gv = jnp.where(jnp.arange(NP) < nvalid, gamma, 0.0).astype(
      jnp.float32)[:, None]
  bvec = jnp.pad(b_i, (0, NP - N))[:, None]
  xf24 = jnp.pad(x.reshape(B * C, HW), ((0, _QW - B * C), (0, 0))).astype(
      jnp.bfloat16)
  xt24 = xf24.T
  blk8 = jnp.tile(
      jnp.concatenate([jnp.repeat(jnp.arange(B, dtype=jnp.int32), C),
                       jnp.full((_QW - B * C,), 99, jnp.int32)])[None, :],
      (8, 1))
  sexp = jnp.concatenate(
      [jnp.pad(jnp.eye(C, dtype=jnp.float32), ((0, _LN - C), (0, 0)))] * B,
      axis=1)
  sexp = jnp.pad(sexp, ((0, 0), (0, _QW - B * C)))
  ssel = sexp.T
  bq16 = jnp.tile(jnp.pad(bq, (0, _LN - C))[None, :], (8, 1))
  bv16 = jnp.tile(jnp.pad(bv, (0, _LN - C))[None, :], (8, 1))
  upd = _tc_flash(xg16, bvec, gv, xf24, xt24, blk8, _padw(Wq.T), _padw(Wk),
                  sexp, ssel, _padw(Wv.T), bq16, bv16, NP, HW)

  supd = jnp.pad(upd, ((0, _NG - NP), (0, 0))).reshape(_NW, _ROWS, 128, _LN)
  sidx = jnp.concatenate(
      [rowidx, BHW + jnp.arange(_NG - N, dtype=jnp.int32) % _SPARE]).reshape(
          _NW, _ROWS, 128)
  y0 = jnp.concatenate(
      [xperm, jnp.zeros((_SPARE, _LN), jnp.float32)])
  yext = _sc_scatter(y0, sidx, supd)
  y = yext[:BHW, :C].reshape(B, HW, C).transpose(0, 2, 1).reshape(B, C, H, W)
  return (y, y)


# speedup vs baseline: 13.1782x; 1.2616x over previous
import functools

import jax
import jax.numpy as jnp
from jax import lax
from jax.experimental import pallas as pl
from jax.experimental.pallas import tpu as pltpu
from jax.experimental.pallas import tpu_sc as plsc
from jax._src.pallas import mpmd as _plmpmd

_NC = 2
_NS = 16
_NW = _NC * _NS
_ROWS = 3
_NG = _NW * _ROWS * 128
_LN = 16
_SPARE = 2304

_NT = 256
_HWT = 3584
_QW = 24


def _sc_mesh():
  return plsc.VectorSubcoreMesh(
      core_axis_name="c", subcore_axis_name="s",
      num_cores=_NC, num_subcores=_NS)


def _sc_gather(table, gidx3):
  @functools.partial(
      pl.kernel,
      out_type=jax.ShapeDtypeStruct((_NW, _ROWS, 128, _LN), jnp.float32),
      mesh=_sc_mesh(),
      compiler_params=pltpu.CompilerParams(use_tc_tiling_on_sc=False),
      scratch_types=[
          pltpu.VMEM((_ROWS, 128), jnp.int32),
          pltpu.VMEM((_ROWS, 128, _LN), jnp.float32),
          pltpu.SemaphoreType.DMA,
      ],
  )
  def gk(table_hbm, idx_hbm, out_hbm, idx_v, rows_v, sem):
    wid = lax.axis_index("s") * _NC + lax.axis_index("c")
    pltpu.sync_copy(idx_hbm.at[wid], idx_v)
    copies = [
        pltpu.async_copy(table_hbm.at[idx_v.at[j]], rows_v.at[j], sem)
        for j in range(_ROWS)
    ]
    for c in copies:
      c.wait()
    pltpu.sync_copy(rows_v, out_hbm.at[wid])

  return gk(table, gidx3)


def _sc_scatter(y0, sidx3, supd3):
  def sk(y_in_hbm, idx_hbm, val_hbm, out_hbm, idx_v, val_v, sem):
    del y_in_hbm
    wid = lax.axis_index("s") * _NC + lax.axis_index("c")
    pltpu.sync_copy(idx_hbm.at[wid], idx_v)
    pltpu.sync_copy(val_hbm.at[wid], val_v)
    copies = [
        pltpu.async_copy(val_v.at[j], out_hbm.at[idx_v.at[j]], sem)
        for j in range(_ROWS)
    ]
    for c in copies:
      c.wait()

  fn = _plmpmd._mpmd_map(
      [(_sc_mesh(), sk)],
      jax.ShapeDtypeStruct(y0.shape, jnp.float32),
      input_output_aliases={0: 0},
      compiler_params=pltpu.CompilerParams(use_tc_tiling_on_sc=False),
      scratch_types=[
          pltpu.VMEM((_ROWS, 128), jnp.int32),
          pltpu.VMEM((_ROWS, 128, _LN), jnp.float32),
          pltpu.SemaphoreType.DMA,
      ],
  )
  return fn(y0, sidx3, supd3)


def _flash_body(xg_ref, b_ref, gv_ref, xf_ref, blk_ref, wqt_ref,
                wk_ref, sexp_ref, ssel_ref, wvt_ref, bq_ref, bv_ref, xb_ref,
                out_ref, qe_ref, m_ref, acc_ref, *, nh):
  j = pl.program_id(1)

  @pl.when(j == 0)
  def _init():
    q = jnp.dot(xg_ref[...], wqt_ref[...],
                preferred_element_type=jnp.float32) + bq_ref[0:1, :]
    qk = jnp.dot(q, wk_ref[...], preferred_element_type=jnp.float32)
    mask = (blk_ref[0:1, :] == b_ref[...]).astype(jnp.float32)
    qe = jnp.dot(qk, sexp_ref[...],
                 preferred_element_type=jnp.float32) * mask
    qe_ref[...] = qe
    m_ref[...] = jnp.sqrt(
        jnp.sum(qe * qe, axis=1, keepdims=True)) * xb_ref[0:1, 0:1]
    acc_ref[...] = jnp.zeros_like(acc_ref[...])

  e = jnp.dot(qe_ref[...].astype(jnp.bfloat16), xf_ref[...],
              preferred_element_type=jnp.float32)
  p = jnp.exp(jnp.maximum(e - m_ref[...], -85.0))
  acc_ref[...] = acc_ref[...] + jax.lax.dot_general(
      p.astype(jnp.bfloat16), xf_ref[...],
      dimension_numbers=(((1,), (1,)), ((), ())),
      preferred_element_type=jnp.float32)

  @pl.when(j == nh - 1)
  def _fin():
    mask = (blk_ref[0:1, :] == b_ref[...]).astype(jnp.float32)
    aexp = acc_ref[...] / acc_ref[...][:, 20:21]
    asel = jnp.dot(aexp * mask, ssel_ref[...],
                   preferred_element_type=jnp.float32)
    outv = jnp.dot(asel, wvt_ref[...],
                   preferred_element_type=jnp.float32) + bv_ref[0:1, :]
    out_ref[...] = xg_ref[...] + gv_ref[...] * outv


def _tc_flash(xg16, bvec, gv, xf24, blk8, wqt, wk, sexp, ssel, wvt,
              bq16, bv16, xb8, np_, hw):
  nn = np_ // _NT
  nh = hw // _HWT
  grid = (nn, nh)
  return pl.pallas_call(
      functools.partial(_flash_body, nh=nh),
      grid=grid,
      in_specs=[
          pl.BlockSpec((_NT, _LN), lambda i, j: (i, 0)),
          pl.BlockSpec((_NT, 1), lambda i, j: (i, 0)),
          pl.BlockSpec((_NT, 1), lambda i, j: (i, 0)),
          pl.BlockSpec((_QW, _HWT), lambda i, j: (0, j)),
          pl.BlockSpec((8, _QW), lambda i, j: (0, 0)),
          pl.BlockSpec((_LN, _LN), lambda i, j: (0, 0)),
          pl.BlockSpec((_LN, _LN), lambda i, j: (0, 0)),
          pl.BlockSpec((_LN, _QW), lambda i, j: (0, 0)),
          pl.BlockSpec((_QW, _LN), lambda i, j: (0, 0)),
          pl.BlockSpec((_LN, _LN), lambda i, j: (0, 0)),
          pl.BlockSpec((8, _LN), lambda i, j: (0, 0)),
          pl.BlockSpec((8, _LN), lambda i, j: (0, 0)),
          pl.BlockSpec((8, 8), lambda i, j: (0, 0)),
      ],
      out_specs=pl.BlockSpec((_NT, _LN), lambda i, j: (i, 0)),
      out_shape=jax.ShapeDtypeStruct((np_, _LN), jnp.float32),
      scratch_shapes=[
          pltpu.VMEM((_NT, _QW), jnp.float32),
          pltpu.VMEM((_NT, 1), jnp.float32),
          pltpu.VMEM((_NT, _QW), jnp.float32),
      ],
  )(xg16, bvec, gv, xf24, blk8, wqt, wk, sexp, ssel, wvt, bq16, bv16, xb8)


def _padw(w):
  return jnp.pad(w, ((0, _LN - w.shape[0]), (0, _LN - w.shape[1])))


def kernel(index, index_len, x, x_teature, gamma, Wq, bq, Wk, bk, Wv, bv):
  del x_teature, bk
  B, C, H, W = x.shape
  HW = H * W
  BHW = B * HW
  N = index.shape[0]
  NP = ((N + _NT - 1) // _NT) * _NT

  b_i = index[:, 0].astype(jnp.int32)
  p_i = (index[:, 1] * W + index[:, 2]).astype(jnp.int32)
  rowidx = b_i * HW + p_i

  xperm = jnp.pad(
      x.reshape(B, C, HW).transpose(0, 2, 1).reshape(BHW, C),
      ((0, 0), (0, _LN - C)))

  gidx = jnp.concatenate(
      [rowidx, jnp.arange(_NG - N, dtype=jnp.int32) % BHW]).reshape(
          _NW, _ROWS, 128)
  xg16 = _sc_gather(xperm, gidx).reshape(_NG, _LN)[:NP]

  nvalid = jnp.minimum(index_len, N)
  gv = jnp.where(jnp.arange(NP) < nvalid, gamma, 0.0).astype(
      jnp.float32)[:, None]
  bvec = jnp.pad(b_i, (0, NP - N))[:, None]
  xf24 = jnp.concatenate([
      x.reshape(B * C, HW).astype(jnp.bfloat16),
      jnp.ones((1, HW), jnp.bfloat16),
      jnp.zeros((_QW - B * C - 1, HW), jnp.bfloat16),
  ])
  xb8 = jnp.full((8, 8), jnp.sqrt(jnp.max(jnp.sum(x * x, axis=1))),
                 jnp.float32)
  blk8 = jnp.tile(
      jnp.concatenate([jnp.repeat(jnp.arange(B, dtype=jnp.int32), C),
                       jnp.full((_QW - B * C,), 99, jnp.int32)])[None, :],
      (8, 1))
  sexp = jnp.concatenate(
      [jnp.pad(jnp.eye(C, dtype=jnp.float32), ((0, _LN - C), (0, 0)))] * B,
      axis=1)
  sexp = jnp.pad(sexp, ((0, 0), (0, _QW - B * C)))
  ssel = sexp.T
  bq16 = jnp.tile(jnp.pad(bq, (0, _LN - C))[None, :], (8, 1))
  bv16 = jnp.tile(jnp.pad(bv, (0, _LN - C))[None, :], (8, 1))
  upd = _tc_flash(xg16, bvec, gv, xf24, blk8, _padw(Wq.T), _padw(Wk),
                  sexp, ssel, _padw(Wv.T), bq16, bv16, xb8, NP, HW)

  supd = jnp.pad(upd, ((0, _NG - NP), (0, 0))).reshape(_NW, _ROWS, 128, _LN)
  sidx = jnp.concatenate(
      [rowidx, BHW + jnp.arange(_NG - N, dtype=jnp.int32) % _SPARE]).reshape(
          _NW, _ROWS, 128)
  y0 = jnp.concatenate(
      [xperm, jnp.zeros((_SPARE, _LN), jnp.float32)])
  yext = _sc_scatter(y0, sidx, supd)
  y = yext[:BHW, :C].reshape(B, HW, C).transpose(0, 2, 1).reshape(B, C, H, W)
  return (y, y)


# speedup vs baseline: 18.5961x; 1.4111x over previous
import functools

import jax
import jax.numpy as jnp
from jax import lax
from jax.experimental import pallas as pl
from jax.experimental.pallas import tpu as pltpu
from jax.experimental.pallas import tpu_sc as plsc
from jax._src.pallas import mpmd as _plmpmd

_NC = 2
_NS = 16
_NW = _NC * _NS
_ROWS = 13
_NG = _NW * _ROWS * 128
_LN = 16
_SPARE = 3328

_NT = 256
_HWT = 3584
_QW = 24


def _sc_mesh():
  return plsc.VectorSubcoreMesh(
      core_axis_name="c", subcore_axis_name="s",
      num_cores=_NC, num_subcores=_NS)


def _sc_gather(table, gidx3):
  @functools.partial(
      pl.kernel,
      out_type=jax.ShapeDtypeStruct((_NW, _ROWS, 128), jnp.float32),
      mesh=_sc_mesh(),
      compiler_params=pltpu.CompilerParams(use_tc_tiling_on_sc=False),
      scratch_types=[
          pltpu.VMEM((_ROWS, 128), jnp.int32),
          pltpu.VMEM((_ROWS, 128), jnp.float32),
          pltpu.SemaphoreType.DMA,
      ],
  )
  def gk(table_hbm, idx_hbm, out_hbm, idx_v, rows_v, sem):
    wid = lax.axis_index("s") * _NC + lax.axis_index("c")
    pltpu.sync_copy(idx_hbm.at[wid], idx_v)
    copies = [
        pltpu.async_copy(table_hbm.at[idx_v.at[j]], rows_v.at[j], sem)
        for j in range(_ROWS)
    ]
    for c in copies:
      c.wait()
    pltpu.sync_copy(rows_v, out_hbm.at[wid])

  return gk(table, gidx3)


def _sc_scatter(y0, sidx3, supd3):
  def sk(y_in_hbm, idx_hbm, val_hbm, out_hbm, idx_v, val_v, sem):
    del y_in_hbm
    wid = lax.axis_index("s") * _NC + lax.axis_index("c")
    pltpu.sync_copy(idx_hbm.at[wid], idx_v)
    pltpu.sync_copy(val_hbm.at[wid], val_v)
    copies = [
        pltpu.async_copy(val_v.at[j], out_hbm.at[idx_v.at[j]], sem)
        for j in range(_ROWS)
    ]
    for c in copies:
      c.wait()

  fn = _plmpmd._mpmd_map(
      [(_sc_mesh(), sk)],
      jax.ShapeDtypeStruct(y0.shape, jnp.float32),
      input_output_aliases={0: 0},
      compiler_params=pltpu.CompilerParams(use_tc_tiling_on_sc=False),
      scratch_types=[
          pltpu.VMEM((_ROWS, 128), jnp.int32),
          pltpu.VMEM((_ROWS, 128), jnp.float32),
          pltpu.SemaphoreType.DMA,
      ],
  )
  return fn(y0, sidx3, supd3)


def _flash_body(xg_ref, b_ref, gv_ref, xf_ref, blk_ref, wqt_ref,
                wk_ref, sexp_ref, ssel_ref, wvt_ref, bq_ref, bv_ref, xb_ref,
                out_ref, qe_ref, m_ref, acc_ref, *, nh):
  j = pl.program_id(1)

  @pl.when(j == 0)
  def _init():
    q = jnp.dot(xg_ref[...], wqt_ref[...],
                preferred_element_type=jnp.float32) + bq_ref[0:1, :]
    qk = jnp.dot(q, wk_ref[...], preferred_element_type=jnp.float32)
    mask = (blk_ref[0:1, :] == b_ref[...]).astype(jnp.float32)
    qe = jnp.dot(qk, sexp_ref[...],
                 preferred_element_type=jnp.float32) * mask
    qe_ref[...] = qe
    m_ref[...] = jnp.sqrt(
        jnp.sum(qe * qe, axis=1, keepdims=True)) * xb_ref[0:1, 0:1]
    acc_ref[...] = jnp.zeros_like(acc_ref[...])

  e = jnp.dot(qe_ref[...].astype(jnp.bfloat16), xf_ref[...],
              preferred_element_type=jnp.float32)
  p = jnp.exp(jnp.maximum(e - m_ref[...], -85.0))
  acc_ref[...] = acc_ref[...] + jax.lax.dot_general(
      p.astype(jnp.bfloat16), xf_ref[...],
      dimension_numbers=(((1,), (1,)), ((), ())),
      preferred_element_type=jnp.float32)

  @pl.when(j == nh - 1)
  def _fin():
    mask = (blk_ref[0:1, :] == b_ref[...]).astype(jnp.float32)
    aexp = acc_ref[...] / acc_ref[...][:, 20:21]
    asel = jnp.dot(aexp * mask, ssel_ref[...],
                   preferred_element_type=jnp.float32)
    outv = jnp.dot(asel, wvt_ref[...],
                   preferred_element_type=jnp.float32) + bv_ref[0:1, :]
    out_ref[...] = xg_ref[...] + gv_ref[...] * outv


def _tc_flash(xg16, bvec, gv, xf24, blk8, wqt, wk, sexp, ssel, wvt,
              bq16, bv16, xb8, np_, hw):
  nn = np_ // _NT
  nh = hw // _HWT
  grid = (nn, nh)
  return pl.pallas_call(
      functools.partial(_flash_body, nh=nh),
      grid=grid,
      in_specs=[
          pl.BlockSpec((_NT, _LN), lambda i, j: (i, 0)),
          pl.BlockSpec((_NT, 1), lambda i, j: (i, 0)),
          pl.BlockSpec((_NT, 1), lambda i, j: (i, 0)),
          pl.BlockSpec((_QW, _HWT), lambda i, j: (0, j)),
          pl.BlockSpec((8, _QW), lambda i, j: (0, 0)),
          pl.BlockSpec((_LN, _LN), lambda i, j: (0, 0)),
          pl.BlockSpec((_LN, _LN), lambda i, j: (0, 0)),
          pl.BlockSpec((_LN, _QW), lambda i, j: (0, 0)),
          pl.BlockSpec((_QW, _LN), lambda i, j: (0, 0)),
          pl.BlockSpec((_LN, _LN), lambda i, j: (0, 0)),
          pl.BlockSpec((8, _LN), lambda i, j: (0, 0)),
          pl.BlockSpec((8, _LN), lambda i, j: (0, 0)),
          pl.BlockSpec((8, 8), lambda i, j: (0, 0)),
      ],
      out_specs=pl.BlockSpec((_NT, _LN), lambda i, j: (i, 0)),
      out_shape=jax.ShapeDtypeStruct((np_, _LN), jnp.float32),
      scratch_shapes=[
          pltpu.VMEM((_NT, _QW), jnp.float32),
          pltpu.VMEM((_NT, 1), jnp.float32),
          pltpu.VMEM((_NT, _QW), jnp.float32),
      ],
  )(xg16, bvec, gv, xf24, blk8, wqt, wk, sexp, ssel, wvt, bq16, bv16, xb8)


def _padw(w):
  return jnp.pad(w, ((0, _LN - w.shape[0]), (0, _LN - w.shape[1])))


def kernel(index, index_len, x, x_teature, gamma, Wq, bq, Wk, bk, Wv, bv):
  del x_teature, bk
  B, C, H, W = x.shape
  HW = H * W
  BCHW = B * C * HW
  N = index.shape[0]
  NP = ((N + _NT - 1) // _NT) * _NT

  b_i = index[:, 0].astype(jnp.int32)
  p_i = (index[:, 1] * W + index[:, 2]).astype(jnp.int32)
  word = (b_i * C)[:, None] * HW + jnp.arange(C, dtype=jnp.int32)[None, :] * HW \
      + p_i[:, None]
  wflat = word.reshape(N * C)

  gidx = jnp.concatenate(
      [wflat, jnp.arange(_NG - N * C, dtype=jnp.int32) % BCHW]).reshape(
          _NW, _ROWS, 128)
  xg_words = _sc_gather(x.reshape(BCHW), gidx).reshape(_NG)
  xg16 = jnp.pad(xg_words[:N * C].reshape(N, C),
                 ((0, NP - N), (0, _LN - C)))

  nvalid = jnp.minimum(index_len, N)
  gv = jnp.where(jnp.arange(NP) < nvalid, gamma, 0.0).astype(
      jnp.float32)[:, None]
  bvec = jnp.pad(b_i, (0, NP - N))[:, None]
  xf24 = jnp.concatenate([
      x.reshape(B * C, HW).astype(jnp.bfloat16),
      jnp.ones((1, HW), jnp.bfloat16),
      jnp.zeros((_QW - B * C - 1, HW), jnp.bfloat16),
  ])
  xb8 = jnp.full((8, 8), jnp.sqrt(jnp.max(jnp.sum(x * x, axis=1))),
                 jnp.float32)
  blk8 = jnp.tile(
      jnp.concatenate([jnp.repeat(jnp.arange(B, dtype=jnp.int32), C),
                       jnp.full((_QW - B * C,), 99, jnp.int32)])[None, :],
      (8, 1))
  sexp = jnp.concatenate(
      [jnp.pad(jnp.eye(C, dtype=jnp.float32), ((0, _LN - C), (0, 0)))] * B,
      axis=1)
  sexp = jnp.pad(sexp, ((0, 0), (0, _QW - B * C)))
  ssel = sexp.T
  bq16 = jnp.tile(jnp.pad(bq, (0, _LN - C))[None, :], (8, 1))
  bv16 = jnp.tile(jnp.pad(bv, (0, _LN - C))[None, :], (8, 1))
  upd = _tc_flash(xg16, bvec, gv, xf24, blk8, _padw(Wq.T), _padw(Wk),
                  sexp, ssel, _padw(Wv.T), bq16, bv16, xb8, NP, HW)

  supd = jnp.concatenate(
      [upd[:, :C].reshape(NP * C),
       jnp.zeros((_NG - NP * C,), jnp.float32)]).reshape(_NW, _ROWS, 128)
  sidx = jnp.concatenate(
      [wflat,
       BCHW + jnp.arange(_NG - N * C, dtype=jnp.int32) % _SPARE]).reshape(
          _NW, _ROWS, 128)
  y0 = jnp.concatenate([x.reshape(BCHW), jnp.zeros((_SPARE,), jnp.float32)])
  yext = _sc_scatter(y0, sidx, supd)
  y = yext[:BCHW].reshape(B, C, H, W)
  return (y, y)


# speedup vs baseline: 21.2441x; 1.1424x over previous
import functools

import jax
import jax.numpy as jnp
from jax import lax
from jax.experimental import pallas as pl
from jax.experimental.pallas import tpu as pltpu
from jax.experimental.pallas import tpu_sc as plsc
from jax._src.pallas import mpmd as _plmpmd

_NC = 2
_NS = 16
_NW = _NC * _NS
_ROWS = 13
_NG = _NW * _ROWS * 128
_LN = 16
_SPARE = 3328

_NT = 512
_HWT = 3584
_QW = 24


def _sc_mesh():
  return plsc.VectorSubcoreMesh(
      core_axis_name="c", subcore_axis_name="s",
      num_cores=_NC, num_subcores=_NS)


def _sc_gather(table, gidx3):
  @functools.partial(
      pl.kernel,
      out_type=jax.ShapeDtypeStruct((_NW, _ROWS, 128), jnp.float32),
      mesh=_sc_mesh(),
      compiler_params=pltpu.CompilerParams(use_tc_tiling_on_sc=False),
      scratch_types=[
          pltpu.VMEM((_ROWS, 128), jnp.int32),
          pltpu.VMEM((_ROWS, 128), jnp.float32),
          pltpu.SemaphoreType.DMA,
      ],
  )
  def gk(table_hbm, idx_hbm, out_hbm, idx_v, rows_v, sem):
    wid = lax.axis_index("s") * _NC + lax.axis_index("c")
    pltpu.sync_copy(idx_hbm.at[wid], idx_v)
    copies = [
        pltpu.async_copy(table_hbm.at[idx_v.at[j]], rows_v.at[j], sem)
        for j in range(_ROWS)
    ]
    for c in copies:
      c.wait()
    pltpu.sync_copy(rows_v, out_hbm.at[wid])

  return gk(table, gidx3)


def _sc_scatter(y0, sidx3, supd3):
  def sk(y_in_hbm, idx_hbm, val_hbm, out_hbm, idx_v, val_v, sem):
    del y_in_hbm
    wid = lax.axis_index("s") * _NC + lax.axis_index("c")
    pltpu.sync_copy(idx_hbm.at[wid], idx_v)
    pltpu.sync_copy(val_hbm.at[wid], val_v)
    copies = [
        pltpu.async_copy(val_v.at[j], out_hbm.at[idx_v.at[j]], sem)
        for j in range(_ROWS)
    ]
    for c in copies:
      c.wait()

  fn = _plmpmd._mpmd_map(
      [(_sc_mesh(), sk)],
      jax.ShapeDtypeStruct(y0.shape, jnp.float32),
      input_output_aliases={0: 0},
      compiler_params=pltpu.CompilerParams(use_tc_tiling_on_sc=False),
      scratch_types=[
          pltpu.VMEM((_ROWS, 128), jnp.int32),
          pltpu.VMEM((_ROWS, 128), jnp.float32),
          pltpu.SemaphoreType.DMA,
      ],
  )
  return fn(y0, sidx3, supd3)


def _flash_body(xg_ref, b_ref, gv_ref, xf_ref, blk_ref, wqt_ref,
                wk_ref, sexp_ref, ssel_ref, wvt_ref, bq_ref, bv_ref, xb_ref,
                out_ref, qe_ref, m_ref, acc_ref, *, nh):
  j = pl.program_id(1)

  @pl.when(j == 0)
  def _init():
    q = jnp.dot(xg_ref[...], wqt_ref[...],
                preferred_element_type=jnp.float32) + bq_ref[0:1, :]
    qk = jnp.dot(q, wk_ref[...], preferred_element_type=jnp.float32)
    mask = (blk_ref[0:1, :] == b_ref[...]).astype(jnp.float32)
    qe = jnp.dot(qk, sexp_ref[...],
                 preferred_element_type=jnp.float32) * mask
    qe_ref[...] = qe
    m_ref[...] = jnp.sqrt(
        jnp.sum(qe * qe, axis=1, keepdims=True)) * xb_ref[0:1, 0:1]
    acc_ref[...] = jnp.zeros_like(acc_ref[...])

  e = jnp.dot(qe_ref[...].astype(jnp.bfloat16), xf_ref[...],
              preferred_element_type=jnp.float32)
  p = jnp.exp(jnp.maximum(e - m_ref[...], -85.0))
  acc_ref[...] = acc_ref[...] + jax.lax.dot_general(
      p.astype(jnp.bfloat16), xf_ref[...],
      dimension_numbers=(((1,), (1,)), ((), ())),
      preferred_element_type=jnp.float32)

  @pl.when(j == nh - 1)
  def _fin():
    mask = (blk_ref[0:1, :] == b_ref[...]).astype(jnp.float32)
    aexp = acc_ref[...] / acc_ref[...][:, 20:21]
    asel = jnp.dot(aexp * mask, ssel_ref[...],
                   preferred_element_type=jnp.float32)
    outv = jnp.dot(asel, wvt_ref[...],
                   preferred_element_type=jnp.float32) + bv_ref[0:1, :]
    out_ref[...] = xg_ref[...] + gv_ref[...] * outv


def _tc_flash(xg16, bvec, gv, xf24, blk8, wqt, wk, sexp, ssel, wvt,
              bq16, bv16, xb8, np_, hw):
  nn = np_ // _NT
  nh = hw // _HWT
  grid = (nn, nh)
  return pl.pallas_call(
      functools.partial(_flash_body, nh=nh),
      grid=grid,
      in_specs=[
          pl.BlockSpec((_NT, _LN), lambda i, j: (i, 0)),
          pl.BlockSpec((_NT, 1), lambda i, j: (i, 0)),
          pl.BlockSpec((_NT, 1), lambda i, j: (i, 0)),
          pl.BlockSpec((_QW, _HWT), lambda i, j: (0, j)),
          pl.BlockSpec((8, _QW), lambda i, j: (0, 0)),
          pl.BlockSpec((_LN, _LN), lambda i, j: (0, 0)),
          pl.BlockSpec((_LN, _LN), lambda i, j: (0, 0)),
          pl.BlockSpec((_LN, _QW), lambda i, j: (0, 0)),
          pl.BlockSpec((_QW, _LN), lambda i, j: (0, 0)),
          pl.BlockSpec((_LN, _LN), lambda i, j: (0, 0)),
          pl.BlockSpec((8, _LN), lambda i, j: (0, 0)),
          pl.BlockSpec((8, _LN), lambda i, j: (0, 0)),
          pl.BlockSpec((8, 8), lambda i, j: (0, 0)),
      ],
      out_specs=pl.BlockSpec((_NT, _LN), lambda i, j: (i, 0)),
      out_shape=jax.ShapeDtypeStruct((np_, _LN), jnp.float32),
      scratch_shapes=[
          pltpu.VMEM((_NT, _QW), jnp.float32),
          pltpu.VMEM((_NT, 1), jnp.float32),
          pltpu.VMEM((_NT, _QW), jnp.float32),
      ],
  )(xg16, bvec, gv, xf24, blk8, wqt, wk, sexp, ssel, wvt, bq16, bv16, xb8)


def _padw(w):
  return jnp.pad(w, ((0, _LN - w.shape[0]), (0, _LN - w.shape[1])))


def kernel(index, index_len, x, x_teature, gamma, Wq, bq, Wk, bk, Wv, bv):
  del x_teature, bk
  B, C, H, W = x.shape
  HW = H * W
  BCHW = B * C * HW
  N = index.shape[0]
  NP = ((N + _NT - 1) // _NT) * _NT

  b_i = index[:, 0].astype(jnp.int32)
  p_i = (index[:, 1] * W + index[:, 2]).astype(jnp.int32)
  word = (b_i * C)[:, None] * HW + jnp.arange(C, dtype=jnp.int32)[None, :] * HW \
      + p_i[:, None]
  wflat = word.reshape(N * C)

  gidx = jnp.concatenate(
      [wflat, jnp.arange(_NG - N * C, dtype=jnp.int32) % BCHW]).reshape(
          _NW, _ROWS, 128)
  xg_words = _sc_gather(x.reshape(BCHW), gidx).reshape(_NG)
  xg16 = jnp.pad(xg_words[:N * C].reshape(N, C),
                 ((0, NP - N), (0, _LN - C)))

  nvalid = jnp.minimum(index_len, N)
  gv = jnp.where(jnp.arange(NP) < nvalid, gamma, 0.0).astype(
      jnp.float32)[:, None]
  bvec = jnp.pad(b_i, (0, NP - N))[:, None]
  xf24 = jnp.concatenate([
      x.reshape(B * C, HW).astype(jnp.bfloat16),
      jnp.ones((1, HW), jnp.bfloat16),
      jnp.zeros((_QW - B * C - 1, HW), jnp.bfloat16),
  ])
  xb8 = jnp.full((8, 8), jnp.sqrt(jnp.max(jnp.sum(x * x, axis=1))),
                 jnp.float32)
  blk8 = jnp.tile(
      jnp.concatenate([jnp.repeat(jnp.arange(B, dtype=jnp.int32), C),
                       jnp.full((_QW - B * C,), 99, jnp.int32)])[None, :],
      (8, 1))
  sexp = jnp.concatenate(
      [jnp.pad(jnp.eye(C, dtype=jnp.float32), ((0, _LN - C), (0, 0)))] * B,
      axis=1)
  sexp = jnp.pad(sexp, ((0, 0), (0, _QW - B * C)))
  ssel = sexp.T
  bq16 = jnp.tile(jnp.pad(bq, (0, _LN - C))[None, :], (8, 1))
  bv16 = jnp.tile(jnp.pad(bv, (0, _LN - C))[None, :], (8, 1))
  upd = _tc_flash(xg16, bvec, gv, xf24, blk8, _padw(Wq.T), _padw(Wk),
                  sexp, ssel, _padw(Wv.T), bq16, bv16, xb8, NP, HW)

  supd = jnp.concatenate(
      [upd[:, :C].reshape(NP * C),
       jnp.zeros((_NG - NP * C,), jnp.float32)]).reshape(_NW, _ROWS, 128)
  sidx = jnp.concatenate(
      [wflat,
       BCHW + jnp.arange(_NG - N * C, dtype=jnp.int32) % _SPARE]).reshape(
          _NW, _ROWS, 128)
  y0 = jnp.concatenate([x.reshape(BCHW), jnp.zeros((_SPARE,), jnp.float32)])
  yext = _sc_scatter(y0, sidx, supd)
  y = yext[:BCHW].reshape(B, C, H, W)
  return (y, y)


# speedup vs baseline: 22.0146x; 1.0363x over previous
import functools

import jax
import jax.numpy as jnp
from jax import lax
from jax.experimental import pallas as pl
from jax.experimental.pallas import tpu as pltpu
from jax.experimental.pallas import tpu_sc as plsc
from jax._src.pallas import mpmd as _plmpmd

_NC = 2
_NS = 16
_NW = _NC * _NS
_ROWS = 13
_NG = _NW * _ROWS * 128
_LN = 16
_SPARE = 3328

_NT = 512
_HWT = 7168
_QW = 24


def _sc_mesh():
  return plsc.VectorSubcoreMesh(
      core_axis_name="c", subcore_axis_name="s",
      num_cores=_NC, num_subcores=_NS)


def _sc_gather(table, gidx3):
  @functools.partial(
      pl.kernel,
      out_type=jax.ShapeDtypeStruct((_NW, _ROWS, 128), jnp.float32),
      mesh=_sc_mesh(),
      compiler_params=pltpu.CompilerParams(use_tc_tiling_on_sc=False),
      scratch_types=[
          pltpu.VMEM((_ROWS, 128), jnp.int32),
          pltpu.VMEM((_ROWS, 128), jnp.float32),
          pltpu.SemaphoreType.DMA,
      ],
  )
  def gk(table_hbm, idx_hbm, out_hbm, idx_v, rows_v, sem):
    wid = lax.axis_index("s") * _NC + lax.axis_index("c")
    pltpu.sync_copy(idx_hbm.at[wid], idx_v)
    copies = [
        pltpu.async_copy(table_hbm.at[idx_v.at[j]], rows_v.at[j], sem)
        for j in range(_ROWS)
    ]
    for c in copies:
      c.wait()
    pltpu.sync_copy(rows_v, out_hbm.at[wid])

  return gk(table, gidx3)


def _sc_scatter(y0, sidx3, supd3):
  def sk(y_in_hbm, idx_hbm, val_hbm, out_hbm, idx_v, val_v, sem):
    del y_in_hbm
    wid = lax.axis_index("s") * _NC + lax.axis_index("c")
    pltpu.sync_copy(idx_hbm.at[wid], idx_v)
    pltpu.sync_copy(val_hbm.at[wid], val_v)
    copies = [
        pltpu.async_copy(val_v.at[j], out_hbm.at[idx_v.at[j]], sem)
        for j in range(_ROWS)
    ]
    for c in copies:
      c.wait()

  fn = _plmpmd._mpmd_map(
      [(_sc_mesh(), sk)],
      jax.ShapeDtypeStruct(y0.shape, jnp.float32),
      input_output_aliases={0: 0},
      compiler_params=pltpu.CompilerParams(use_tc_tiling_on_sc=False),
      scratch_types=[
          pltpu.VMEM((_ROWS, 128), jnp.int32),
          pltpu.VMEM((_ROWS, 128), jnp.float32),
          pltpu.SemaphoreType.DMA,
      ],
  )
  return fn(y0, sidx3, supd3)


def _flash_body(xg_ref, b_ref, gv_ref, xf_ref, blk_ref, wqt_ref,
                wk_ref, sexp_ref, ssel_ref, wvt_ref, bq_ref, bv_ref, xb_ref,
                out_ref, qe_ref, m_ref, acc_ref, *, nh):
  j = pl.program_id(1)

  @pl.when(j == 0)
  def _init():
    q = jnp.dot(xg_ref[...], wqt_ref[...],
                preferred_element_type=jnp.float32) + bq_ref[0:1, :]
    qk = jnp.dot(q, wk_ref[...], preferred_element_type=jnp.float32)
    mask = (blk_ref[0:1, :] == b_ref[...]).astype(jnp.float32)
    qe = jnp.dot(qk, sexp_ref[...],
                 preferred_element_type=jnp.float32) * mask
    qe_ref[...] = qe
    m_ref[...] = jnp.sqrt(
        jnp.sum(qe * qe, axis=1, keepdims=True)) * xb_ref[0:1, 0:1]
    acc_ref[...] = jnp.zeros_like(acc_ref[...])

  e = jnp.dot(qe_ref[...].astype(jnp.bfloat16), xf_ref[...],
              preferred_element_type=jnp.float32)
  p = jnp.exp(jnp.maximum(e - m_ref[...], -85.0))
  acc_ref[...] = acc_ref[...] + jax.lax.dot_general(
      p.astype(jnp.bfloat16), xf_ref[...],
      dimension_numbers=(((1,), (1,)), ((), ())),
      preferred_element_type=jnp.float32)

  @pl.when(j == nh - 1)
  def _fin():
    mask = (blk_ref[0:1, :] == b_ref[...]).astype(jnp.float32)
    aexp = acc_ref[...] / acc_ref[...][:, 20:21]
    asel = jnp.dot(aexp * mask, ssel_ref[...],
                   preferred_element_type=jnp.float32)
    outv = jnp.dot(asel, wvt_ref[...],
                   preferred_element_type=jnp.float32) + bv_ref[0:1, :]
    out_ref[...] = xg_ref[...] + gv_ref[...] * outv


def _tc_flash(xg16, bvec, gv, xf24, blk8, wqt, wk, sexp, ssel, wvt,
              bq16, bv16, xb8, np_, hw):
  nn = np_ // _NT
  nh = hw // _HWT
  grid = (nn, nh)
  return pl.pallas_call(
      functools.partial(_flash_body, nh=nh),
      grid=grid,
      in_specs=[
          pl.BlockSpec((_NT, _LN), lambda i, j: (i, 0)),
          pl.BlockSpec((_NT, 1), lambda i, j: (i, 0)),
          pl.BlockSpec((_NT, 1), lambda i, j: (i, 0)),
          pl.BlockSpec((_QW, _HWT), lambda i, j: (0, j)),
          pl.BlockSpec((8, _QW), lambda i, j: (0, 0)),
          pl.BlockSpec((_LN, _LN), lambda i, j: (0, 0)),
          pl.BlockSpec((_LN, _LN), lambda i, j: (0, 0)),
          pl.BlockSpec((_LN, _QW), lambda i, j: (0, 0)),
          pl.BlockSpec((_QW, _LN), lambda i, j: (0, 0)),
          pl.BlockSpec((_LN, _LN), lambda i, j: (0, 0)),
          pl.BlockSpec((8, _LN), lambda i, j: (0, 0)),
          pl.BlockSpec((8, _LN), lambda i, j: (0, 0)),
          pl.BlockSpec((8, 8), lambda i, j: (0, 0)),
      ],
      out_specs=pl.BlockSpec((_NT, _LN), lambda i, j: (i, 0)),
      out_shape=jax.ShapeDtypeStruct((np_, _LN), jnp.float32),
      scratch_shapes=[
          pltpu.VMEM((_NT, _QW), jnp.float32),
          pltpu.VMEM((_NT, 1), jnp.float32),
          pltpu.VMEM((_NT, _QW), jnp.float32),
      ],
  )(xg16, bvec, gv, xf24, blk8, wqt, wk, sexp, ssel, wvt, bq16, bv16, xb8)


def _padw(w):
  return jnp.pad(w, ((0, _LN - w.shape[0]), (0, _LN - w.shape[1])))


def kernel(index, index_len, x, x_teature, gamma, Wq, bq, Wk, bk, Wv, bv):
  del x_teature, bk
  B, C, H, W = x.shape
  HW = H * W
  BCHW = B * C * HW
  N = index.shape[0]
  NP = ((N + _NT - 1) // _NT) * _NT

  b_i = index[:, 0].astype(jnp.int32)
  p_i = (index[:, 1] * W + index[:, 2]).astype(jnp.int32)
  word = (b_i * C)[:, None] * HW + jnp.arange(C, dtype=jnp.int32)[None, :] * HW \
      + p_i[:, None]
  wflat = word.reshape(N * C)

  gidx = jnp.concatenate(
      [wflat, jnp.arange(_NG - N * C, dtype=jnp.int32) % BCHW]).reshape(
          _NW, _ROWS, 128)
  xg_words = _sc_gather(x.reshape(BCHW), gidx).reshape(_NG)
  xg16 = jnp.pad(xg_words[:N * C].reshape(N, C),
                 ((0, NP - N), (0, _LN - C)))

  nvalid = jnp.minimum(index_len, N)
  gv = jnp.where(jnp.arange(NP) < nvalid, gamma, 0.0).astype(
      jnp.float32)[:, None]
  bvec = jnp.pad(b_i, (0, NP - N))[:, None]
  xf24 = jnp.concatenate([
      x.reshape(B * C, HW).astype(jnp.bfloat16),
      jnp.ones((1, HW), jnp.bfloat16),
      jnp.zeros((_QW - B * C - 1, HW), jnp.bfloat16),
  ])
  xb8 = jnp.full((8, 8), jnp.sqrt(jnp.max(jnp.sum(x * x, axis=1))),
                 jnp.float32)
  blk8 = jnp.tile(
      jnp.concatenate([jnp.repeat(jnp.arange(B, dtype=jnp.int32), C),
                       jnp.full((_QW - B * C,), 99, jnp.int32)])[None, :],
      (8, 1))
  sexp = jnp.concatenate(
      [jnp.pad(jnp.eye(C, dtype=jnp.float32), ((0, _LN - C), (0, 0)))] * B,
      axis=1)
  sexp = jnp.pad(sexp, ((0, 0), (0, _QW - B * C)))
  ssel = sexp.T
  bq16 = jnp.tile(jnp.pad(bq, (0, _LN - C))[None, :], (8, 1))
  bv16 = jnp.tile(jnp.pad(bv, (0, _LN - C))[None, :], (8, 1))
  upd = _tc_flash(xg16, bvec, gv, xf24, blk8, _padw(Wq.T), _padw(Wk),
                  sexp, ssel, _padw(Wv.T), bq16, bv16, xb8, NP, HW)

  supd = jnp.concatenate(
      [upd[:, :C].reshape(NP * C),
       jnp.zeros((_NG - NP * C,), jnp.float32)]).reshape(_NW, _ROWS, 128)
  sidx = jnp.concatenate(
      [wflat,
       BCHW + jnp.arange(_NG - N * C, dtype=jnp.int32) % _SPARE]).reshape(
          _NW, _ROWS, 128)
  y0 = jnp.concatenate([x.reshape(BCHW), jnp.zeros((_SPARE,), jnp.float32)])
  yext = _sc_scatter(y0, sidx, supd)
  y = yext[:BCHW].reshape(B, C, H, W)
  return (y, y)


# speedup vs baseline: 24.5530x; 1.1153x over previous
import functools

import jax
import jax.numpy as jnp
from jax import lax
from jax.experimental import pallas as pl
from jax.experimental.pallas import tpu as pltpu
from jax.experimental.pallas import tpu_sc as plsc
from jax._src.pallas import mpmd as _plmpmd

_NC = 2
_NS = 16
_NW = _NC * _NS
_ROWS = 13
_NG = _NW * _ROWS * 128
_LN = 16
_SPARE = 3328

_NT = 1024
_HWT = 7168
_QW = 24


def _sc_mesh():
  return plsc.VectorSubcoreMesh(
      core_axis_name="c", subcore_axis_name="s",
      num_cores=_NC, num_subcores=_NS)


def _sc_gather(table, gidx3):
  @functools.partial(
      pl.kernel,
      out_type=jax.ShapeDtypeStruct((_NW, _ROWS, 128), jnp.float32),
      mesh=_sc_mesh(),
      compiler_params=pltpu.CompilerParams(use_tc_tiling_on_sc=False),
      scratch_types=[
          pltpu.VMEM((_ROWS, 128), jnp.int32),
          pltpu.VMEM((_ROWS, 128), jnp.float32),
          pltpu.SemaphoreType.DMA,
      ],
  )
  def gk(table_hbm, idx_hbm, out_hbm, idx_v, rows_v, sem):
    wid = lax.axis_index("s") * _NC + lax.axis_index("c")
    pltpu.sync_copy(idx_hbm.at[wid], idx_v)
    copies = [
        pltpu.async_copy(table_hbm.at[idx_v.at[j]], rows_v.at[j], sem)
        for j in range(_ROWS)
    ]
    for c in copies:
      c.wait()
    pltpu.sync_copy(rows_v, out_hbm.at[wid])

  return gk(table, gidx3)


def _sc_scatter(y0, sidx3, supd3):
  def sk(y_in_hbm, idx_hbm, val_hbm, out_hbm, idx_v, val_v, sem):
    del y_in_hbm
    wid = lax.axis_index("s") * _NC + lax.axis_index("c")
    pltpu.sync_copy(idx_hbm.at[wid], idx_v)
    pltpu.sync_copy(val_hbm.at[wid], val_v)
    copies = [
        pltpu.async_copy(val_v.at[j], out_hbm.at[idx_v.at[j]], sem)
        for j in range(_ROWS)
    ]
    for c in copies:
      c.wait()

  fn = _plmpmd._mpmd_map(
      [(_sc_mesh(), sk)],
      jax.ShapeDtypeStruct(y0.shape, jnp.float32),
      input_output_aliases={0: 0},
      compiler_params=pltpu.CompilerParams(use_tc_tiling_on_sc=False),
      scratch_types=[
          pltpu.VMEM((_ROWS, 128), jnp.int32),
          pltpu.VMEM((_ROWS, 128), jnp.float32),
          pltpu.SemaphoreType.DMA,
      ],
  )
  return fn(y0, sidx3, supd3)


def _flash_body(xg_ref, b_ref, gv_ref, xf_ref, blk_ref, wqt_ref,
                wk_ref, sexp_ref, ssel_ref, wvt_ref, bq_ref, bv_ref, xb_ref,
                out_ref, qe_ref, acc_ref, *, nh):
  j = pl.program_id(1)

  @pl.when(j == 0)
  def _init():
    q = jnp.dot(xg_ref[...], wqt_ref[...],
                preferred_element_type=jnp.float32) + bq_ref[0:1, :]
    qk = jnp.dot(q, wk_ref[...], preferred_element_type=jnp.float32)
    mask = (blk_ref[0:1, :] == b_ref[...]).astype(jnp.float32)
    qe = jnp.dot(qk, sexp_ref[...],
                 preferred_element_type=jnp.float32) * mask
    qe = qe * 1.4426950408889634
    m = jnp.sqrt(jnp.sum(qe * qe, axis=1, keepdims=True)) * xb_ref[0:1, 0:1]
    oneh = (blk_ref[0:1, :] == 100).astype(jnp.float32)
    qe_ref[...] = qe - m * oneh
    acc_ref[...] = jnp.zeros_like(acc_ref[...])

  e = jnp.dot(qe_ref[...].astype(jnp.bfloat16), xf_ref[...],
              preferred_element_type=jnp.float32)
  p = jnp.exp2(e)
  acc_ref[...] = acc_ref[...] + jax.lax.dot_general(
      p.astype(jnp.bfloat16), xf_ref[...],
      dimension_numbers=(((1,), (1,)), ((), ())),
      preferred_element_type=jnp.float32)

  @pl.when(j == nh - 1)
  def _fin():
    mask = (blk_ref[0:1, :] == b_ref[...]).astype(jnp.float32)
    aexp = acc_ref[...] / (acc_ref[...][:, 20:21] + 1e-30)
    asel = jnp.dot(aexp * mask, ssel_ref[...],
                   preferred_element_type=jnp.float32)
    outv = jnp.dot(asel, wvt_ref[...],
                   preferred_element_type=jnp.float32) + bv_ref[0:1, :]
    out_ref[...] = xg_ref[...] + gv_ref[...] * outv


def _tc_flash(xg16, bvec, gv, xf24, blk8, wqt, wk, sexp, ssel, wvt,
              bq16, bv16, xb8, np_, hw):
  nn = np_ // _NT
  nh = hw // _HWT
  grid = (nn, nh)
  return pl.pallas_call(
      functools.partial(_flash_body, nh=nh),
      grid=grid,
      in_specs=[
          pl.BlockSpec((_NT, _LN), lambda i, j: (i, 0)),
          pl.BlockSpec((_NT, 1), lambda i, j: (i, 0)),
          pl.BlockSpec((_NT, 1), lambda i, j: (i, 0)),
          pl.BlockSpec((_QW, _HWT), lambda i, j: (0, j)),
          pl.BlockSpec((8, _QW), lambda i, j: (0, 0)),
          pl.BlockSpec((_LN, _LN), lambda i, j: (0, 0)),
          pl.BlockSpec((_LN, _LN), lambda i, j: (0, 0)),
          pl.BlockSpec((_LN, _QW), lambda i, j: (0, 0)),
          pl.BlockSpec((_QW, _LN), lambda i, j: (0, 0)),
          pl.BlockSpec((_LN, _LN), lambda i, j: (0, 0)),
          pl.BlockSpec((8, _LN), lambda i, j: (0, 0)),
          pl.BlockSpec((8, _LN), lambda i, j: (0, 0)),
          pl.BlockSpec((8, 8), lambda i, j: (0, 0)),
      ],
      out_specs=pl.BlockSpec((_NT, _LN), lambda i, j: (i, 0)),
      out_shape=jax.ShapeDtypeStruct((np_, _LN), jnp.float32),
      scratch_shapes=[
          pltpu.VMEM((_NT, _QW), jnp.float32),
          pltpu.VMEM((_NT, _QW), jnp.float32),
      ],
  )(xg16, bvec, gv, xf24, blk8, wqt, wk, sexp, ssel, wvt, bq16, bv16, xb8)


def _padw(w):
  return jnp.pad(w, ((0, _LN - w.shape[0]), (0, _LN - w.shape[1])))


def kernel(index, index_len, x, x_teature, gamma, Wq, bq, Wk, bk, Wv, bv):
  del x_teature, bk
  B, C, H, W = x.shape
  HW = H * W
  BCHW = B * C * HW
  N = index.shape[0]
  NP = ((N + _NT - 1) // _NT) * _NT

  b_i = index[:, 0].astype(jnp.int32)
  p_i = (index[:, 1] * W + index[:, 2]).astype(jnp.int32)
  word = (b_i * C)[:, None] * HW + jnp.arange(C, dtype=jnp.int32)[None, :] * HW \
      + p_i[:, None]
  wflat = word.reshape(N * C)

  gidx = jnp.concatenate(
      [wflat, jnp.arange(_NG - N * C, dtype=jnp.int32) % BCHW]).reshape(
          _NW, _ROWS, 128)
  xg_words = _sc_gather(x.reshape(BCHW), gidx).reshape(_NG)
  xg16 = jnp.pad(xg_words[:N * C].reshape(N, C),
                 ((0, NP - N), (0, _LN - C)))

  nvalid = jnp.minimum(index_len, N)
  gv = jnp.where(jnp.arange(NP) < nvalid, gamma, 0.0).astype(
      jnp.float32)[:, None]
  bvec = jnp.pad(b_i, (0, NP - N))[:, None]
  xf24 = jnp.concatenate([
      x.reshape(B * C, HW).astype(jnp.bfloat16),
      jnp.ones((1, HW), jnp.bfloat16),
      jnp.zeros((_QW - B * C - 1, HW), jnp.bfloat16),
  ])
  xb8 = jnp.full((8, 8), jnp.sqrt(jnp.max(jnp.sum(x * x, axis=1))),
                 jnp.float32)
  blk8 = jnp.tile(
      jnp.concatenate([jnp.repeat(jnp.arange(B, dtype=jnp.int32), C),
                       jnp.array([100], jnp.int32),
                       jnp.full((_QW - B * C - 1,), 99, jnp.int32)])[None, :],
      (8, 1))
  sexp = jnp.concatenate(
      [jnp.pad(jnp.eye(C, dtype=jnp.float32), ((0, _LN - C), (0, 0)))] * B,
      axis=1)
  sexp = jnp.pad(sexp, ((0, 0), (0, _QW - B * C)))
  ssel = sexp.T
  bq16 = jnp.tile(jnp.pad(bq, (0, _LN - C))[None, :], (8, 1))
  bv16 = jnp.tile(jnp.pad(bv, (0, _LN - C))[None, :], (8, 1))
  upd = _tc_flash(xg16, bvec, gv, xf24, blk8, _padw(Wq.T), _padw(Wk),
                  sexp, ssel, _padw(Wv.T), bq16, bv16, xb8, NP, HW)

  updflat = upd[:, :C].reshape(NP * C)[:N * C]
  supd = jnp.concatenate(
      [updflat, updflat[:_NG - N * C]]).reshape(_NW, _ROWS, 128)
  sidx = jnp.concatenate(
      [wflat, wflat[:_NG - N * C]]).reshape(_NW, _ROWS, 128)
  yext = _sc_scatter(x.reshape(BCHW), sidx, supd)
  y = yext.reshape(B, C, H, W)
  return (y, y)
